# baseline (speedup 1.0000x reference)
import jax
import jax.numpy as jnp
from jax import lax
from jax.experimental import pallas as pl
from jax.experimental.pallas import tpu as pltpu
from jax.experimental.pallas import tpu_sc as plsc

_B, _T, _D, _H, _DH = 1, 2048, 768, 12, 64
_V, _E, _EDIM, _K = 50304, 16, 512, 2
_TB = 256
_NTB = _T // _TB

_NC, _NS = 2, 16
_NW = _NC * _NS
_GCHUNK = 64

_BLK = 256
_NPAIR = _T * _K
_G = _NPAIR // _BLK + _E
_NP = _G * _BLK


def _row_gather(table, idx_flat, nrows):
    rpw = nrows // _NW
    nch = rpw // _GCHUNK

    def body(table_hbm, idx_hbm, out_hbm, idx_v, rows_v, sem):
        wid = lax.axis_index("s") * _NC + lax.axis_index("c")
        base = wid * rpw
        for c in range(nch):
            off = base + c * _GCHUNK
            pltpu.sync_copy(idx_hbm.at[pl.ds(off, _GCHUNK)], idx_v)
            pltpu.async_copy(table_hbm.at[idx_v], rows_v, sem).wait()
            pltpu.sync_copy(rows_v, out_hbm.at[pl.ds(off, _GCHUNK)])

    mesh = plsc.VectorSubcoreMesh(core_axis_name="c", subcore_axis_name="s")
    k = pl.kernel(
        body,
        mesh=mesh,
        out_type=jax.ShapeDtypeStruct((nrows, _D), jnp.float32),
        scratch_types=[
            pltpu.VMEM((_GCHUNK,), jnp.int32),
            pltpu.VMEM((_GCHUNK, _D), jnp.float32),
            pltpu.SemaphoreType.DMA,
        ],
    )
    return k(table, idx_flat)


def _recip(d):
    r = 1.0 / d
    r = r * (2.0 - d * r)
    r = r * (2.0 - d * r)
    return r


def _rsqrt_acc(v):
    r = lax.rsqrt(v)
    r = r * (1.5 - 0.5 * v * r * r)
    r = r * (1.5 - 0.5 * v * r * r)
    return r


def _ln(h, w, b):
    mu = jnp.mean(h, axis=-1, keepdims=True)
    var = jnp.mean((h - mu) ** 2, axis=-1, keepdims=True)
    return (h - mu) * _rsqrt_acc(var + 1e-5) * w + b


def _gelu(x):
    return 0.5 * x * (1.0 + lax.erf(x * 0.7071067811865476))


def _split3(a):
    hi = a.astype(jnp.bfloat16)
    lo = (a - hi.astype(jnp.float32)).astype(jnp.bfloat16)
    return hi, lo


def _dot3(a, b, dims):
    ah, al = _split3(a)
    bh, bl = _split3(b)
    f = jnp.float32
    return (lax.dot_general(ah, bl, dims, preferred_element_type=f)
            + lax.dot_general(al, bh, dims, preferred_element_type=f)
            + lax.dot_general(ah, bh, dims, preferred_element_type=f))


def _rmant(x, m):
    u = lax.bitcast_convert_type(x, jnp.int32)
    s = 23 - m
    bias = (1 << (s - 1)) - 1
    u2 = u + bias + ((u >> s) & 1)
    u2 = u2 & ~((1 << s) - 1)
    return lax.bitcast_convert_type(u2, jnp.float32)


def _dot_tf32(a, b, dims, m=10):
    return lax.dot_general(_rmant(a, m), _rmant(b, m), dims,
                           precision=lax.Precision.HIGHEST,
                           preferred_element_type=jnp.float32)


_DT = (((1,), (1,)), ((), ()))
_DN = (((1,), (0,)), ((), ()))


def _dot_t(a, b):
    return lax.dot_general(a, b, (((1,), (1,)), ((), ())),
                           preferred_element_type=jnp.float32)


def _exp_acc(x):
    xc = jnp.maximum(x, -88.723164)
    m = jnp.floor(xc * 1.44269504088896341 + 0.5)
    r = m * (-0.693359375) + xc
    r = m * 2.12194440e-4 + r
    r2 = r * r
    p = 1.9875691500e-4
    p = p * r + 1.3981999507e-3
    p = p * r + 8.3334519073e-3
    p = p * r + 4.1665795894e-2
    p = p * r + 1.6666665459e-1
    p = p * r + 5.0000001201e-1
    p = p * r2 + r + 1.0
    mi = jnp.maximum(m, -126.0).astype(jnp.int32)
    sc = lax.bitcast_convert_type((mi + 127) << 23, jnp.float32)
    return jnp.where(x < -87.0, 0.0, p * sc)


def _preattn_body(emb_ref, pos_ref, w1_ref, b1_ref, wattn_ref, x_ref, qkv_ref):
    x = emb_ref[...] + pos_ref[...]
    x_ref[...] = x
    h = _ln(x, w1_ref[...], b1_ref[...])
    qkv_ref[...] = _dot_t(h, wattn_ref[...])


def _attn_body(qkv_ref, y_ref):
    i = pl.program_id(0)
    scale = 1.0 / jnp.sqrt(jnp.float32(_DH))
    rows = i * _TB + lax.broadcasted_iota(jnp.int32, (_TB, _T), 0)
    cols = lax.broadcasted_iota(jnp.int32, (_TB, _T), 1)
    mask = cols <= rows
    neg = jnp.finfo(jnp.float32).min
    for h in range(_H):
        q = qkv_ref[pl.ds(i * _TB, _TB), h * _DH:(h + 1) * _DH]
        kk = qkv_ref[:, _D + h * _DH:_D + (h + 1) * _DH]
        v = qkv_ref[:, 2 * _D + h * _DH:2 * _D + (h + 1) * _DH]
        s = _dot_t(q, kk) * scale
        s = jnp.where(mask, s, neg)
        m = jnp.max(s, axis=-1, keepdims=True)
        p = _exp_acc(s - m)
        p = p * _recip(jnp.sum(p, axis=-1, keepdims=True))
        y_ref[:, h * _DH:(h + 1) * _DH] = lax.dot_general(
            p, v, _DN, preferred_element_type=jnp.float32)


def _postattn_body(y_ref, x_ref, w2_ref, b2_ref, wproj_ref, wshup_ref,
                   wshdn_ref, wrout_ref, h2_ref, base_ref, topi_ref,
                   topp_ref):
    x = x_ref[...]
    attn = _dot_t(y_ref[...], wproj_ref[...]) + x
    h2 = _ln(attn, w2_ref[...], b2_ref[...])
    h2_ref[...] = h2
    up = _dot_t(h2, wshup_ref[...])
    shared = _dot_t(_gelu(up), wshdn_ref[...])
    base_ref[...] = shared + x
    logits = _dot_t(h2, wrout_ref[...])
    lm = jnp.max(logits, axis=-1, keepdims=True)
    pe = _exp_acc(logits - lm)
    probs = pe * _recip(jnp.sum(pe, axis=-1, keepdims=True))
    col = lax.broadcasted_iota(jnp.int32, (_TB, _E), 1)
    m1 = jnp.max(probs, axis=-1, keepdims=True)
    i1 = jnp.min(jnp.where(probs == m1, col, _E), axis=-1, keepdims=True)
    p2 = jnp.where(col == i1, -1.0, probs)
    m2 = jnp.max(p2, axis=-1, keepdims=True)
    i2 = jnp.min(jnp.where((probs == m2) & (col != i1), col, _E),
                 axis=-1, keepdims=True)
    topi_ref[...] = jnp.concatenate([i1, i2], axis=1)
    topp_ref[...] = jnp.concatenate([m1, m2], axis=1)


def _expert_body(bexp_ref, xg_ref, wup_ref, wdn_ref, ys_ref):
    del bexp_ref
    up = _dot_t(xg_ref[...], wup_ref[0])
    ys_ref[...] = _dot_t(_gelu(up), wdn_ref[0])


def _combine_body(base_ref, yg_ref, topp_ref, out_ref):
    tp = topp_ref[...]
    out_ref[...] = (base_ref[...]
                    + yg_ref[:, :_D] * tp[:, 0:1]
                    + yg_ref[:, _D:] * tp[:, 1:2])


def _dispatch_meta(topi):
    tf = topi.reshape(_NPAIR)
    oh = (tf[:, None] == jnp.arange(_E, dtype=jnp.int32)[None, :]).astype(jnp.int32)
    cs = jnp.cumsum(oh, axis=0)
    rank = jnp.take_along_axis(cs, tf[:, None], axis=1)[:, 0] - 1
    counts = cs[-1]
    pc = ((counts + _BLK - 1) // _BLK) * _BLK
    po = jnp.concatenate([jnp.zeros(1, jnp.int32), jnp.cumsum(pc)])[:_E]
    pp = (po[tf] + rank).astype(jnp.int32)
    tok_padded = jnp.zeros(_NP, jnp.int32).at[pp].set(
        jnp.arange(_NPAIR, dtype=jnp.int32) // _K)
    bexp = jnp.searchsorted(jnp.cumsum(pc // _BLK),
                            jnp.arange(_G), side='right').astype(jnp.int32)
    bexp = jnp.minimum(bexp, _E - 1)
    return tok_padded, pp, bexp


def kernel(idx, token_emb, pos_emb, ln1_w, ln1_b, ln2_w, ln2_b, W_attn,
           W_proj, W_router, W_sh_up, W_sh_down, W_up, W_down):
    idx_flat = idx.reshape(_T).astype(jnp.int32)
    emb = _row_gather(token_emb, idx_flat, _T)

    x, qkv = pl.pallas_call(
        _preattn_body,
        grid=(_NTB,),
        in_specs=[
            pl.BlockSpec((_TB, _D), lambda i: (i, 0)),
            pl.BlockSpec((_TB, _D), lambda i: (i, 0)),
            pl.BlockSpec((1, _D), lambda i: (0, 0)),
            pl.BlockSpec((1, _D), lambda i: (0, 0)),
            pl.BlockSpec((3 * _D, _D), lambda i: (0, 0)),
        ],
        out_specs=[
            pl.BlockSpec((_TB, _D), lambda i: (i, 0)),
            pl.BlockSpec((_TB, 3 * _D), lambda i: (i, 0)),
        ],
        out_shape=[
            jax.ShapeDtypeStruct((_T, _D), jnp.float32),
            jax.ShapeDtypeStruct((_T, 3 * _D), jnp.float32),
        ],
    )(emb, pos_emb, ln1_w.reshape(1, _D), ln1_b.reshape(1, _D), W_attn)

    y = pl.pallas_call(
        _attn_body,
        grid=(_NTB,),
        in_specs=[pl.BlockSpec((_T, 3 * _D), lambda i: (0, 0))],
        out_specs=pl.BlockSpec((_TB, _D), lambda i: (i, 0)),
        out_shape=jax.ShapeDtypeStruct((_T, _D), jnp.float32),
    )(qkv)

    h2, base, topi, topp = pl.pallas_call(
        _postattn_body,
        grid=(_NTB,),
        in_specs=[
            pl.BlockSpec((_TB, _D), lambda i: (i, 0)),
            pl.BlockSpec((_TB, _D), lambda i: (i, 0)),
            pl.BlockSpec((1, _D), lambda i: (0, 0)),
            pl.BlockSpec((1, _D), lambda i: (0, 0)),
            pl.BlockSpec((_D, _D), lambda i: (0, 0)),
            pl.BlockSpec((_EDIM, _D), lambda i: (0, 0)),
            pl.BlockSpec((_D, _EDIM), lambda i: (0, 0)),
            pl.BlockSpec((_E, _D), lambda i: (0, 0)),
        ],
        out_specs=[
            pl.BlockSpec((_TB, _D), lambda i: (i, 0)),
            pl.BlockSpec((_TB, _D), lambda i: (i, 0)),
            pl.BlockSpec((_TB, _K), lambda i: (i, 0)),
            pl.BlockSpec((_TB, _K), lambda i: (i, 0)),
        ],
        out_shape=[
            jax.ShapeDtypeStruct((_T, _D), jnp.float32),
            jax.ShapeDtypeStruct((_T, _D), jnp.float32),
            jax.ShapeDtypeStruct((_T, _K), jnp.int32),
            jax.ShapeDtypeStruct((_T, _K), jnp.float32),
        ],
    )(y, x, ln2_w.reshape(1, _D), ln2_b.reshape(1, _D), W_proj, W_sh_up,
      W_sh_down, W_router)

    tok_padded, pp, bexp = _dispatch_meta(topi)

    xg = _row_gather(h2, tok_padded, _NP)

    ys = pl.pallas_call(
        _expert_body,
        grid_spec=pltpu.PrefetchScalarGridSpec(
            num_scalar_prefetch=1,
            grid=(_G,),
            in_specs=[
                pl.BlockSpec((_BLK, _D), lambda g, b: (g, 0)),
                pl.BlockSpec((1, _EDIM, _D), lambda g, b: (b[g], 0, 0)),
                pl.BlockSpec((1, _D, _EDIM), lambda g, b: (b[g], 0, 0)),
            ],
            out_specs=pl.BlockSpec((_BLK, _D), lambda g, b: (g, 0)),
        ),
        out_shape=jax.ShapeDtypeStruct((_NP, _D), jnp.float32),
    )(bexp, xg, W_up, W_down)

    yg = _row_gather(ys, pp, _NPAIR).reshape(_T, _K * _D)

    out = pl.pallas_call(
        _combine_body,
        grid=(_NTB,),
        in_specs=[
            pl.BlockSpec((_TB, _D), lambda i: (i, 0)),
            pl.BlockSpec((_TB, _K * _D), lambda i: (i, 0)),
            pl.BlockSpec((_TB, _K), lambda i: (i, 0)),
        ],
        out_specs=pl.BlockSpec((_TB, _D), lambda i: (i, 0)),
        out_shape=jax.ShapeDtypeStruct((_T, _D), jnp.float32),
    )(base, yg, topp)

    return out.reshape(_B, _T, _D)


# speedup vs baseline: 1.4854x; 1.4854x over previous
import jax
import jax.numpy as jnp
from jax import lax
from jax.experimental import pallas as pl
from jax.experimental.pallas import tpu as pltpu
from jax.experimental.pallas import tpu_sc as plsc

_B, _T, _D, _H, _DH = 1, 2048, 768, 12, 64
_V, _E, _EDIM, _K = 50304, 16, 512, 2
_TB = 256
_NTB = _T // _TB

_NC, _NS = 2, 16
_NW = _NC * _NS
_GCHUNK = 64

_BLK = 256
_NPAIR = _T * _K
_G = _NPAIR // _BLK + _E
_NP = _G * _BLK


def _row_gather(table, idx_flat, nrows):
    rpw = nrows // _NW
    nch = rpw // _GCHUNK

    def body(table_hbm, idx_hbm, out_hbm, idx_v, rows_v, sem):
        wid = lax.axis_index("s") * _NC + lax.axis_index("c")
        base = wid * rpw
        for c in range(nch):
            off = base + c * _GCHUNK
            pltpu.sync_copy(idx_hbm.at[pl.ds(off, _GCHUNK)], idx_v)
            pltpu.async_copy(table_hbm.at[idx_v], rows_v, sem).wait()
            pltpu.sync_copy(rows_v, out_hbm.at[pl.ds(off, _GCHUNK)])

    mesh = plsc.VectorSubcoreMesh(core_axis_name="c", subcore_axis_name="s")
    k = pl.kernel(
        body,
        mesh=mesh,
        out_type=jax.ShapeDtypeStruct((nrows, _D), jnp.float32),
        scratch_types=[
            pltpu.VMEM((_GCHUNK,), jnp.int32),
            pltpu.VMEM((_GCHUNK, _D), jnp.float32),
            pltpu.SemaphoreType.DMA,
        ],
    )
    return k(table, idx_flat)


def _recip(d):
    r = 1.0 / d
    r = r * (2.0 - d * r)
    r = r * (2.0 - d * r)
    return r


def _rsqrt_acc(v):
    r = lax.rsqrt(v)
    r = r * (1.5 - 0.5 * v * r * r)
    r = r * (1.5 - 0.5 * v * r * r)
    return r


def _ln(h, w, b):
    mu = jnp.mean(h, axis=-1, keepdims=True)
    var = jnp.mean((h - mu) ** 2, axis=-1, keepdims=True)
    return (h - mu) * _rsqrt_acc(var + 1e-5) * w + b


def _gelu(x):
    return 0.5 * x * (1.0 + lax.erf(x * 0.7071067811865476))


def _split3(a):
    hi = a.astype(jnp.bfloat16)
    lo = (a - hi.astype(jnp.float32)).astype(jnp.bfloat16)
    return hi, lo


def _dot3(a, b, dims):
    ah, al = _split3(a)
    bh, bl = _split3(b)
    f = jnp.float32
    return (lax.dot_general(ah, bl, dims, preferred_element_type=f)
            + lax.dot_general(al, bh, dims, preferred_element_type=f)
            + lax.dot_general(ah, bh, dims, preferred_element_type=f))


def _rmant(x, m):
    u = lax.bitcast_convert_type(x, jnp.int32)
    s = 23 - m
    bias = (1 << (s - 1)) - 1
    u2 = u + bias + ((u >> s) & 1)
    u2 = u2 & ~((1 << s) - 1)
    return lax.bitcast_convert_type(u2, jnp.float32)


def _dot_tf32(a, b, dims, m=10):
    return lax.dot_general(_rmant(a, m), _rmant(b, m), dims,
                           precision=lax.Precision.HIGHEST,
                           preferred_element_type=jnp.float32)


_DT = (((1,), (1,)), ((), ()))
_DN = (((1,), (0,)), ((), ()))


def _dot_t(a, b):
    return lax.dot_general(a, b, (((1,), (1,)), ((), ())),
                           preferred_element_type=jnp.float32)


def _exp_acc(x):
    xc = jnp.maximum(x, -88.723164)
    m = jnp.floor(xc * 1.44269504088896341 + 0.5)
    r = m * (-0.693359375) + xc
    r = m * 2.12194440e-4 + r
    r2 = r * r
    p = 1.9875691500e-4
    p = p * r + 1.3981999507e-3
    p = p * r + 8.3334519073e-3
    p = p * r + 4.1665795894e-2
    p = p * r + 1.6666665459e-1
    p = p * r + 5.0000001201e-1
    p = p * r2 + r + 1.0
    mi = jnp.maximum(m, -126.0).astype(jnp.int32)
    sc = lax.bitcast_convert_type((mi + 127) << 23, jnp.float32)
    return jnp.where(x < -87.0, 0.0, p * sc)


def _preattn_body(emb_ref, pos_ref, w1_ref, b1_ref, wattn_ref, x_ref, qkv_ref):
    x = emb_ref[...] + pos_ref[...]
    x_ref[...] = x
    h = _ln(x, w1_ref[...], b1_ref[...])
    qkv_ref[...] = _dot_t(h, wattn_ref[...])


def _attn_body(qkv_ref, y_ref):
    i = pl.program_id(0)
    scale = 1.0 / jnp.sqrt(jnp.float32(_DH))
    rows = i * _TB + lax.broadcasted_iota(jnp.int32, (_TB, _T), 0)
    cols = lax.broadcasted_iota(jnp.int32, (_TB, _T), 1)
    mask = cols <= rows
    neg = jnp.finfo(jnp.float32).min
    for h in range(_H):
        q = qkv_ref[pl.ds(i * _TB, _TB), h * _DH:(h + 1) * _DH]
        kk = qkv_ref[:, _D + h * _DH:_D + (h + 1) * _DH]
        v = qkv_ref[:, 2 * _D + h * _DH:2 * _D + (h + 1) * _DH]
        s = _dot_t(q, kk) * scale
        s = jnp.where(mask, s, neg)
        m = jnp.max(s, axis=-1, keepdims=True)
        p = _exp_acc(s - m)
        p = p * _recip(jnp.sum(p, axis=-1, keepdims=True))
        y_ref[:, h * _DH:(h + 1) * _DH] = lax.dot_general(
            p, v, _DN, preferred_element_type=jnp.float32)


def _postattn_body(y_ref, x_ref, w2_ref, b2_ref, wproj_ref, wshup_ref,
                   wshdn_ref, wrout_ref, h2_ref, base_ref, gate_ref):
    x = x_ref[...]
    attn = _dot_t(y_ref[...], wproj_ref[...]) + x
    h2 = _ln(attn, w2_ref[...], b2_ref[...])
    h2_ref[...] = h2
    up = _dot_t(h2, wshup_ref[...])
    shared = _dot_t(_gelu(up), wshdn_ref[...])
    base_ref[...] = shared + x
    logits = _dot_t(h2, wrout_ref[...])
    lm = jnp.max(logits, axis=-1, keepdims=True)
    pe = _exp_acc(logits - lm)
    probs = pe * _recip(jnp.sum(pe, axis=-1, keepdims=True))
    col = lax.broadcasted_iota(jnp.int32, (_TB, _E), 1)
    m1 = jnp.max(probs, axis=-1, keepdims=True)
    i1 = jnp.min(jnp.where(probs == m1, col, _E), axis=-1, keepdims=True)
    p2 = jnp.where(col == i1, -1.0, probs)
    m2 = jnp.max(p2, axis=-1, keepdims=True)
    i2 = jnp.min(jnp.where((probs == m2) & (col != i1), col, _E),
                 axis=-1, keepdims=True)
    gate_ref[...] = jnp.where((col == i1) | (col == i2), probs, 0.0)


def _moe_body(h2_ref, base_ref, gate_ref, wup_ref, wdn_ref, out_ref):
    e = pl.program_id(0)

    @pl.when(e == 0)
    def _():
        out_ref[...] = base_ref[...]

    emask = (lax.broadcasted_iota(jnp.int32, (1, _E), 1) == e).astype(jnp.float32)
    gcol = jnp.sum(gate_ref[...] * emask, axis=1, keepdims=True)
    up = _dot_t(h2_ref[...], wup_ref[0])
    dn = _dot_t(_gelu(up), wdn_ref[0])
    out_ref[...] += dn * gcol


def kernel(idx, token_emb, pos_emb, ln1_w, ln1_b, ln2_w, ln2_b, W_attn,
           W_proj, W_router, W_sh_up, W_sh_down, W_up, W_down):
    idx_flat = idx.reshape(_T).astype(jnp.int32)
    emb = _row_gather(token_emb, idx_flat, _T)

    x, qkv = pl.pallas_call(
        _preattn_body,
        grid=(_NTB,),
        in_specs=[
            pl.BlockSpec((_TB, _D), lambda i: (i, 0)),
            pl.BlockSpec((_TB, _D), lambda i: (i, 0)),
            pl.BlockSpec((1, _D), lambda i: (0, 0)),
            pl.BlockSpec((1, _D), lambda i: (0, 0)),
            pl.BlockSpec((3 * _D, _D), lambda i: (0, 0)),
        ],
        out_specs=[
            pl.BlockSpec((_TB, _D), lambda i: (i, 0)),
            pl.BlockSpec((_TB, 3 * _D), lambda i: (i, 0)),
        ],
        out_shape=[
            jax.ShapeDtypeStruct((_T, _D), jnp.float32),
            jax.ShapeDtypeStruct((_T, 3 * _D), jnp.float32),
        ],
    )(emb, pos_emb, ln1_w.reshape(1, _D), ln1_b.reshape(1, _D), W_attn)

    y = pl.pallas_call(
        _attn_body,
        grid=(_NTB,),
        in_specs=[pl.BlockSpec((_T, 3 * _D), lambda i: (0, 0))],
        out_specs=pl.BlockSpec((_TB, _D), lambda i: (i, 0)),
        out_shape=jax.ShapeDtypeStruct((_T, _D), jnp.float32),
    )(qkv)

    h2, base, gate = pl.pallas_call(
        _postattn_body,
        grid=(_NTB,),
        in_specs=[
            pl.BlockSpec((_TB, _D), lambda i: (i, 0)),
            pl.BlockSpec((_TB, _D), lambda i: (i, 0)),
            pl.BlockSpec((1, _D), lambda i: (0, 0)),
            pl.BlockSpec((1, _D), lambda i: (0, 0)),
            pl.BlockSpec((_D, _D), lambda i: (0, 0)),
            pl.BlockSpec((_EDIM, _D), lambda i: (0, 0)),
            pl.BlockSpec((_D, _EDIM), lambda i: (0, 0)),
            pl.BlockSpec((_E, _D), lambda i: (0, 0)),
        ],
        out_specs=[
            pl.BlockSpec((_TB, _D), lambda i: (i, 0)),
            pl.BlockSpec((_TB, _D), lambda i: (i, 0)),
            pl.BlockSpec((_TB, _E), lambda i: (i, 0)),
        ],
        out_shape=[
            jax.ShapeDtypeStruct((_T, _D), jnp.float32),
            jax.ShapeDtypeStruct((_T, _D), jnp.float32),
            jax.ShapeDtypeStruct((_T, _E), jnp.float32),
        ],
    )(y, x, ln2_w.reshape(1, _D), ln2_b.reshape(1, _D), W_proj, W_sh_up,
      W_sh_down, W_router)

    out = pl.pallas_call(
        _moe_body,
        grid=(_E,),
        in_specs=[
            pl.BlockSpec((_T, _D), lambda e: (0, 0)),
            pl.BlockSpec((_T, _D), lambda e: (0, 0)),
            pl.BlockSpec((_T, _E), lambda e: (0, 0)),
            pl.BlockSpec((1, _EDIM, _D), lambda e: (e, 0, 0)),
            pl.BlockSpec((1, _D, _EDIM), lambda e: (e, 0, 0)),
        ],
        out_specs=pl.BlockSpec((_T, _D), lambda e: (0, 0)),
        out_shape=jax.ShapeDtypeStruct((_T, _D), jnp.float32),
        compiler_params=pltpu.CompilerParams(
            dimension_semantics=("arbitrary",)),
    )(h2, base, gate, W_up, W_down)

    return out.reshape(_B, _T, _D)


# speedup vs baseline: 3.1478x; 2.1192x over previous
import jax
import jax.numpy as jnp
from jax import lax
from jax.experimental import pallas as pl
from jax.experimental.pallas import tpu as pltpu
from jax.experimental.pallas import tpu_sc as plsc

_B, _T, _D, _H, _DH = 1, 2048, 768, 12, 64
_V, _E, _EDIM, _K = 50304, 16, 512, 2
_TB = 256
_NTB = _T // _TB

_NC, _NS = 2, 16
_NW = _NC * _NS
_GCHUNK = 64

_BLK = 256
_NPAIR = _T * _K
_G = _NPAIR // _BLK + _E
_NP = _G * _BLK


def _row_gather(table, idx_flat, nrows):
    rpw = nrows // _NW
    nch = rpw // _GCHUNK

    def body(table_hbm, idx_hbm, out_hbm, idx_v, rows_v, sem):
        wid = lax.axis_index("s") * _NC + lax.axis_index("c")
        base = wid * rpw
        for c in range(nch):
            off = base + c * _GCHUNK
            pltpu.sync_copy(idx_hbm.at[pl.ds(off, _GCHUNK)], idx_v)
            pltpu.async_copy(table_hbm.at[idx_v], rows_v, sem).wait()
            pltpu.sync_copy(rows_v, out_hbm.at[pl.ds(off, _GCHUNK)])

    mesh = plsc.VectorSubcoreMesh(core_axis_name="c", subcore_axis_name="s")
    k = pl.kernel(
        body,
        mesh=mesh,
        out_type=jax.ShapeDtypeStruct((nrows, _D), jnp.float32),
        scratch_types=[
            pltpu.VMEM((_GCHUNK,), jnp.int32),
            pltpu.VMEM((_GCHUNK, _D), jnp.float32),
            pltpu.SemaphoreType.DMA,
        ],
    )
    return k(table, idx_flat)


def _recip(d):
    r = 1.0 / d
    r = r * (2.0 - d * r)
    r = r * (2.0 - d * r)
    return r


def _rsqrt_acc(v):
    r = lax.rsqrt(v)
    r = r * (1.5 - 0.5 * v * r * r)
    r = r * (1.5 - 0.5 * v * r * r)
    return r


def _ln(h, w, b):
    mu = jnp.mean(h, axis=-1, keepdims=True)
    var = jnp.mean((h - mu) ** 2, axis=-1, keepdims=True)
    return (h - mu) * lax.rsqrt(var + 1e-5) * w + b


def _gelu(x):
    return 0.5 * x * (1.0 + lax.erf(x * 0.7071067811865476))


def _split3(a):
    hi = a.astype(jnp.bfloat16)
    lo = (a - hi.astype(jnp.float32)).astype(jnp.bfloat16)
    return hi, lo


def _dot3(a, b, dims):
    ah, al = _split3(a)
    bh, bl = _split3(b)
    f = jnp.float32
    return (lax.dot_general(ah, bl, dims, preferred_element_type=f)
            + lax.dot_general(al, bh, dims, preferred_element_type=f)
            + lax.dot_general(ah, bh, dims, preferred_element_type=f))


def _rmant(x, m):
    u = lax.bitcast_convert_type(x, jnp.int32)
    s = 23 - m
    bias = (1 << (s - 1)) - 1
    u2 = u + bias + ((u >> s) & 1)
    u2 = u2 & ~((1 << s) - 1)
    return lax.bitcast_convert_type(u2, jnp.float32)


def _dot_tf32(a, b, dims, m=10):
    return lax.dot_general(_rmant(a, m), _rmant(b, m), dims,
                           precision=lax.Precision.HIGHEST,
                           preferred_element_type=jnp.float32)


_DT = (((1,), (1,)), ((), ()))
_DN = (((1,), (0,)), ((), ()))


def _dot_t(a, b):
    return lax.dot_general(a, b, (((1,), (1,)), ((), ())),
                           preferred_element_type=jnp.float32)


def _exp_acc(x):
    xc = jnp.maximum(x, -88.723164)
    m = jnp.floor(xc * 1.44269504088896341 + 0.5)
    r = m * (-0.693359375) + xc
    r = m * 2.12194440e-4 + r
    r2 = r * r
    p = 1.9875691500e-4
    p = p * r + 1.3981999507e-3
    p = p * r + 8.3334519073e-3
    p = p * r + 4.1665795894e-2
    p = p * r + 1.6666665459e-1
    p = p * r + 5.0000001201e-1
    p = p * r2 + r + 1.0
    mi = jnp.maximum(m, -126.0).astype(jnp.int32)
    sc = lax.bitcast_convert_type((mi + 127) << 23, jnp.float32)
    return jnp.where(x < -87.0, 0.0, p * sc)


def _preattn_body(emb_ref, pos_ref, w1_ref, b1_ref, wattn_ref, x_ref, qkv_ref):
    x = emb_ref[...] + pos_ref[...]
    x_ref[...] = x
    h = _ln(x, w1_ref[...], b1_ref[...])
    qkv_ref[...] = _dot_t(h, wattn_ref[...])


def _attn_body(qkv_ref, y_ref):
    i = pl.program_id(0)
    scale = 1.0 / jnp.sqrt(jnp.float32(_DH))
    rows = i * _TB + lax.broadcasted_iota(jnp.int32, (_TB, _T), 0)
    cols = lax.broadcasted_iota(jnp.int32, (_TB, _T), 1)
    mask = cols <= rows
    neg = jnp.finfo(jnp.float32).min
    for h in range(_H):
        q = qkv_ref[pl.ds(i * _TB, _TB), h * _DH:(h + 1) * _DH]
        kk = qkv_ref[:, _D + h * _DH:_D + (h + 1) * _DH]
        v = qkv_ref[:, 2 * _D + h * _DH:2 * _D + (h + 1) * _DH]
        s = _dot_t(q, kk) * scale
        s = jnp.where(mask, s, neg)
        m = jnp.max(s, axis=-1, keepdims=True)
        p = jnp.exp(s - m)
        p = p / jnp.sum(p, axis=-1, keepdims=True)
        y_ref[:, h * _DH:(h + 1) * _DH] = lax.dot_general(
            p, v, _DN, preferred_element_type=jnp.float32)


def _postattn_body(y_ref, x_ref, w2_ref, b2_ref, wproj_ref, wshup_ref,
                   wshdn_ref, wrout_ref, h2_ref, base_ref, gate_ref):
    x = x_ref[...]
    attn = _dot_t(y_ref[...], wproj_ref[...]) + x
    h2 = _ln(attn, w2_ref[...], b2_ref[...])
    h2_ref[...] = h2
    up = _dot_t(h2, wshup_ref[...])
    shared = _dot_t(_gelu(up), wshdn_ref[...])
    base_ref[...] = shared + x
    logits = _dot_t(h2, wrout_ref[...])
    lm = jnp.max(logits, axis=-1, keepdims=True)
    pe = jnp.exp(logits - lm)
    probs = pe / jnp.sum(pe, axis=-1, keepdims=True)
    col = lax.broadcasted_iota(jnp.int32, (_TB, _E), 1)
    m1 = jnp.max(probs, axis=-1, keepdims=True)
    i1 = jnp.min(jnp.where(probs == m1, col, _E), axis=-1, keepdims=True)
    p2 = jnp.where(col == i1, -1.0, probs)
    m2 = jnp.max(p2, axis=-1, keepdims=True)
    i2 = jnp.min(jnp.where((probs == m2) & (col != i1), col, _E),
                 axis=-1, keepdims=True)
    gate_ref[...] = jnp.where((col == i1) | (col == i2), probs, 0.0)


def _moe_body(h2_ref, base_ref, gate_ref, wup_ref, wdn_ref, out_ref):
    e = pl.program_id(0)

    @pl.when(e == 0)
    def _():
        out_ref[...] = base_ref[...]

    emask = (lax.broadcasted_iota(jnp.int32, (1, _E), 1) == e).astype(jnp.float32)
    gcol = jnp.sum(gate_ref[...] * emask, axis=1, keepdims=True)
    up = _dot_t(h2_ref[...], wup_ref[0])
    dn = _dot_t(_gelu(up), wdn_ref[0])
    out_ref[...] += dn * gcol


def kernel(idx, token_emb, pos_emb, ln1_w, ln1_b, ln2_w, ln2_b, W_attn,
           W_proj, W_router, W_sh_up, W_sh_down, W_up, W_down):
    idx_flat = idx.reshape(_T).astype(jnp.int32)
    emb = _row_gather(token_emb, idx_flat, _T)

    x, qkv = pl.pallas_call(
        _preattn_body,
        grid=(_NTB,),
        in_specs=[
            pl.BlockSpec((_TB, _D), lambda i: (i, 0)),
            pl.BlockSpec((_TB, _D), lambda i: (i, 0)),
            pl.BlockSpec((1, _D), lambda i: (0, 0)),
            pl.BlockSpec((1, _D), lambda i: (0, 0)),
            pl.BlockSpec((3 * _D, _D), lambda i: (0, 0)),
        ],
        out_specs=[
            pl.BlockSpec((_TB, _D), lambda i: (i, 0)),
            pl.BlockSpec((_TB, 3 * _D), lambda i: (i, 0)),
        ],
        out_shape=[
            jax.ShapeDtypeStruct((_T, _D), jnp.float32),
            jax.ShapeDtypeStruct((_T, 3 * _D), jnp.float32),
        ],
    )(emb, pos_emb, ln1_w.reshape(1, _D), ln1_b.reshape(1, _D), W_attn)

    y = pl.pallas_call(
        _attn_body,
        grid=(_NTB,),
        in_specs=[pl.BlockSpec((_T, 3 * _D), lambda i: (0, 0))],
        out_specs=pl.BlockSpec((_TB, _D), lambda i: (i, 0)),
        out_shape=jax.ShapeDtypeStruct((_T, _D), jnp.float32),
    )(qkv)

    h2, base, gate = pl.pallas_call(
        _postattn_body,
        grid=(_NTB,),
        in_specs=[
            pl.BlockSpec((_TB, _D), lambda i: (i, 0)),
            pl.BlockSpec((_TB, _D), lambda i: (i, 0)),
            pl.BlockSpec((1, _D), lambda i: (0, 0)),
            pl.BlockSpec((1, _D), lambda i: (0, 0)),
            pl.BlockSpec((_D, _D), lambda i: (0, 0)),
            pl.BlockSpec((_EDIM, _D), lambda i: (0, 0)),
            pl.BlockSpec((_D, _EDIM), lambda i: (0, 0)),
            pl.BlockSpec((_E, _D), lambda i: (0, 0)),
        ],
        out_specs=[
            pl.BlockSpec((_TB, _D), lambda i: (i, 0)),
            pl.BlockSpec((_TB, _D), lambda i: (i, 0)),
            pl.BlockSpec((_TB, _E), lambda i: (i, 0)),
        ],
        out_shape=[
            jax.ShapeDtypeStruct((_T, _D), jnp.float32),
            jax.ShapeDtypeStruct((_T, _D), jnp.float32),
            jax.ShapeDtypeStruct((_T, _E), jnp.float32),
        ],
    )(y, x, ln2_w.reshape(1, _D), ln2_b.reshape(1, _D), W_proj, W_sh_up,
      W_sh_down, W_router)

    out = pl.pallas_call(
        _moe_body,
        grid=(_E,),
        in_specs=[
            pl.BlockSpec((_T, _D), lambda e: (0, 0)),
            pl.BlockSpec((_T, _D), lambda e: (0, 0)),
            pl.BlockSpec((_T, _E), lambda e: (0, 0)),
            pl.BlockSpec((1, _EDIM, _D), lambda e: (e, 0, 0)),
            pl.BlockSpec((1, _D, _EDIM), lambda e: (e, 0, 0)),
        ],
        out_specs=pl.BlockSpec((_T, _D), lambda e: (0, 0)),
        out_shape=jax.ShapeDtypeStruct((_T, _D), jnp.float32),
        compiler_params=pltpu.CompilerParams(
            dimension_semantics=("arbitrary",)),
    )(h2, base, gate, W_up, W_down)

    return out.reshape(_B, _T, _D)
